# baseline (speedup 1.0000x reference)
import functools
import math

import jax
import jax.numpy as jnp
from jax.experimental import pallas as pl
from jax.experimental.pallas import tpu as pltpu

D_, H_, DFF_, PRED_ = 768, 12, 2048, 24
FACTOR_ = 5
DK_ = D_ // H_
SCALE_ = 1.0 / math.sqrt(DK_)
UPAD_ = 40


def _split(w):
    w_hi = w.astype(jnp.bfloat16)
    w_lo = (w - w_hi.astype(jnp.float32)).astype(jnp.bfloat16)
    return w_hi, w_lo


def _d3(ah, al, wh, wl):
    d = lambda p, q: jnp.dot(p, q, preferred_element_type=jnp.float32)
    return d(ah, wh) + (d(ah, wl) + d(al, wh))


def _asplit(a):
    ah = a.astype(jnp.bfloat16)
    al = (a - ah.astype(jnp.float32)).astype(jnp.bfloat16)
    return ah, al


def _embed_qkv_kernel(x_ref, w_ref, b_ref, pos_ref, wq_ref, bq_ref,
                      h_ref, qkv_ref):
    h = (jnp.dot(x_ref[...], w_ref[...], preferred_element_type=jnp.float32)
         + b_ref[...] + pos_ref[...])
    h_ref[...] = h
    qkv_ref[...] = (jnp.dot(h, wq_ref[...], preferred_element_type=jnp.float32)
                    + bq_ref[...])


def _embed_qkv(x2, w, b, pos, wqkv, bqkv, tm=1024):
    M, F = x2.shape
    D = w.shape[1]
    N = wqkv.shape[1]
    tm = min(tm, pos.shape[0])
    nl = pos.shape[0] // tm
    return pl.pallas_call(
        _embed_qkv_kernel,
        grid=(M // tm,),
        in_specs=[
            pl.BlockSpec((tm, F), lambda m: (m, 0)),
            pl.BlockSpec((F, D), lambda m: (0, 0)),
            pl.BlockSpec((1, D), lambda m: (0, 0)),
            pl.BlockSpec((tm, D), lambda m: (m % nl, 0)),
            pl.BlockSpec((D, N), lambda m: (0, 0)),
            pl.BlockSpec((1, N), lambda m: (0, 0)),
        ],
        out_specs=[
            pl.BlockSpec((tm, D), lambda m: (m, 0)),
            pl.BlockSpec((tm, N), lambda m: (m, 0)),
        ],
        out_shape=[
            jax.ShapeDtypeStruct((M, D), jnp.float32),
            jax.ShapeDtypeStruct((M, N), jnp.float32),
        ],
    )(x2, w, b, pos, wqkv, bqkv)


def _mm_kernel(a_ref, w_ref, b_ref, o_ref):
    o_ref[...] = (
        jnp.dot(a_ref[...], w_ref[...], preferred_element_type=jnp.float32)
        + b_ref[...])


def _mm(a, w, b, tm=1024):
    M, K = a.shape
    N = w.shape[1]
    tm = min(tm, M)
    return pl.pallas_call(
        _mm_kernel,
        grid=(M // tm,),
        in_specs=[
            pl.BlockSpec((tm, K), lambda m: (m, 0)),
            pl.BlockSpec((K, N), lambda m: (0, 0)),
            pl.BlockSpec((1, N), lambda m: (0, 0)),
        ],
        out_specs=pl.BlockSpec((tm, N), lambda m: (m, 0)),
        out_shape=jax.ShapeDtypeStruct((M, N), jnp.float32),
    )(a, w, b)


def _block_kernel(c_ref, r_ref, wo_ref, bo_ref, g1_ref, be1_ref,
                  w1_ref, b1_ref, w2_ref, b2_ref, g2_ref, be2_ref, o_ref):
    y = (jnp.dot(c_ref[...], wo_ref[...], preferred_element_type=jnp.float32)
         + bo_ref[...] + r_ref[...])
    mu = jnp.mean(y, axis=-1, keepdims=True)
    d = y - mu
    va = jnp.mean(d * d, axis=-1, keepdims=True)
    h1 = d / jnp.sqrt(va + 1e-5) * g1_ref[...] + be1_ref[...]
    t = (jnp.dot(h1, w1_ref[...], preferred_element_type=jnp.float32)
         + b1_ref[...])
    t = 0.5 * t * (1.0 + jax.lax.erf(t * (1.0 / math.sqrt(2.0))))
    y2 = (jnp.dot(t, w2_ref[...], preferred_element_type=jnp.float32)
          + b2_ref[...] + h1)
    mu2 = jnp.mean(y2, axis=-1, keepdims=True)
    d2 = y2 - mu2
    va2 = jnp.mean(d2 * d2, axis=-1, keepdims=True)
    o_ref[...] = d2 / jnp.sqrt(va2 + 1e-5) * g2_ref[...] + be2_ref[...]


def _block(c, r, lp, tm=1024):
    M, K = c.shape
    N = lp["W1"].shape[1]
    tm = min(tm, M)
    full = lambda m: (0, 0)
    row = lambda m: (m, 0)
    return pl.pallas_call(
        _block_kernel,
        grid=(M // tm,),
        in_specs=[
            pl.BlockSpec((tm, K), row),
            pl.BlockSpec((tm, K), row),
            pl.BlockSpec((K, K), full),
            pl.BlockSpec((1, K), full),
            pl.BlockSpec((1, K), full),
            pl.BlockSpec((1, K), full),
            pl.BlockSpec((K, N), full),
            pl.BlockSpec((1, N), full),
            pl.BlockSpec((N, K), full),
            pl.BlockSpec((1, K), full),
            pl.BlockSpec((1, K), full),
            pl.BlockSpec((1, K), full),
        ],
        out_specs=pl.BlockSpec((tm, K), row),
        out_shape=jax.ShapeDtypeStruct((M, K), jnp.float32),
    )(c, r, lp["Wo"], lp["bo"].reshape(1, -1),
      lp["g1"].reshape(1, -1), lp["be1"].reshape(1, -1),
      lp["W1"], lp["b1"].reshape(1, -1), lp["W2"], lp["b2"].reshape(1, -1),
      lp["g2"].reshape(1, -1), lp["be2"].reshape(1, -1))


OHP_ = 48


def _select_kernel(idx_ref, q_ref, k_ref, oh_ref, m_ref, *, u, lq):
    idxcol = idx_ref[...]
    lanes_u = jax.lax.broadcasted_iota(jnp.int32, (UPAD_, lq), 1)
    oh_idx = jnp.where(lanes_u == idxcol, SCALE_, 0.0)

    for h in range(H_):
        q = q_ref[0][:, h * DK_:(h + 1) * DK_]
        k = k_ref[0][:, h * DK_:(h + 1) * DK_]
        ksamp = jnp.dot(oh_idx, k, preferred_element_type=jnp.float32)
        st = jax.lax.dot_general(
            ksamp, q, (((1,), (1,)), ((), ())),
            preferred_element_type=jnp.float32)
        if u < UPAD_:
            rows = jax.lax.broadcasted_iota(jnp.int32, (UPAD_, lq), 0)
            smax = jnp.max(jnp.where(rows < u, st, -jnp.inf), 0, keepdims=True)
            smean = (jnp.sum(jnp.where(rows < u, st, 0.0), 0, keepdims=True)
                     * (1.0 / u))
        else:
            smax = jnp.max(st, axis=0, keepdims=True)
            smean = jnp.sum(st, axis=0, keepdims=True) * (1.0 / u)
        m_ref[h:h + 1, :] = smax - smean

    m_ref[H_:, :] = jnp.full((16 - H_, lq), -jnp.inf, jnp.float32)
    mall = m_ref[...]
    lanes16 = jax.lax.broadcasted_iota(jnp.int32, (16, lq), 1)

    for j in range(u):
        mx = jnp.max(mall, axis=1, keepdims=True)
        i = jnp.min(jnp.where(mall == mx, lanes16, lq), axis=1, keepdims=True)
        ohj = lanes16 == i
        fj = ohj.astype(jnp.float32)
        for h in range(H_):
            oh_ref[0, h, j:j + 1, :] = fj[h:h + 1, :]
        mall = jnp.where(ohj, -jnp.inf, mall)

    ones_row = jnp.ones((1, lq), jnp.float32)
    zeros_tail = jnp.zeros((OHP_ - u - 1, lq), jnp.float32)
    for h in range(H_):
        oh_ref[0, h, u:u + 1, :] = ones_row
        oh_ref[0, h, u + 1:, :] = zeros_tail


def _apply_kernel(q_ref, k_ref, v_ref, oh_ref, o_ref, *, u, lq):
    for t, off in enumerate((0, DK_)):
        q = q_ref[0][:, off:off + DK_]
        k = k_ref[0][:, off:off + DK_]
        v = v_ref[0][:, off:off + DK_]
        oh = oh_ref[0, t]

        qtop = jnp.dot(oh, q, preferred_element_type=jnp.float32) * SCALE_
        s = jax.lax.dot_general(
            qtop, k, (((1,), (1,)), ((), ())),
            preferred_element_type=jnp.float32)
        s = s - jnp.max(s, axis=1, keepdims=True)
        e = jnp.exp(s)
        denom = jnp.dot(e, jnp.ones((lq, 1), jnp.float32),
                        preferred_element_type=jnp.float32)
        p = e / denom
        ctx_top = jnp.dot(p, v, preferred_element_type=jnp.float32)

        mv = jnp.dot(jnp.full((1, lq), 1.0 / lq, jnp.float32), v,
                     preferred_element_type=jnp.float32)
        rows_c = jax.lax.broadcasted_iota(jnp.int32, (OHP_, DK_), 0)
        ctx_aug = jnp.where(rows_c == u, mv, ctx_top - mv)
        o_ref[0, :, off:off + DK_] = jax.lax.dot_general(
            oh, ctx_aug, (((0,), (0,)), ((), ())),
            preferred_element_type=jnp.float32)


def _attn(qkv, idx_k, u):
    Bq, Lq, _ = qkv.shape
    hp = H_ // 2
    idx_pad = jnp.full((UPAD_, 1), -1, jnp.int32).at[:u, 0].set(idx_k)
    oh = pl.pallas_call(
        functools.partial(_select_kernel, u=u, lq=Lq),
        grid=(Bq,),
        in_specs=[
            pl.BlockSpec((UPAD_, 1), lambda b: (0, 0)),
            pl.BlockSpec((1, Lq, D_), lambda b: (b, 0, 0)),
            pl.BlockSpec((1, Lq, D_), lambda b: (b, 0, 1)),
        ],
        out_specs=pl.BlockSpec((1, H_, OHP_, Lq), lambda b: (b, 0, 0, 0)),
        scratch_shapes=[pltpu.VMEM((16, Lq), jnp.float32)],
        out_shape=jax.ShapeDtypeStruct((Bq, H_, OHP_, Lq), jnp.float32),
    )(idx_pad, qkv, qkv)
    return pl.pallas_call(
        functools.partial(_apply_kernel, u=u, lq=Lq),
        grid=(Bq, hp),
        in_specs=[
            pl.BlockSpec((1, Lq, 2 * DK_), lambda b, h: (b, 0, h)),
            pl.BlockSpec((1, Lq, 2 * DK_), lambda b, h: (b, 0, hp + h)),
            pl.BlockSpec((1, Lq, 2 * DK_), lambda b, h: (b, 0, 2 * hp + h)),
            pl.BlockSpec((1, 2, OHP_, Lq), lambda b, h: (b, h, 0, 0)),
        ],
        out_specs=pl.BlockSpec((1, Lq, 2 * DK_), lambda b, h: (b, 0, h)),
        out_shape=jax.ShapeDtypeStruct((Bq, Lq, D_), jnp.float32),
    )(qkv, qkv, qkv, oh)


def _attn_last_kernel(idx_ref, q_ref, k_ref, v_ref, o_ref, *, u, lq):
    idxcol = idx_ref[...]
    lanes_u = jax.lax.broadcasted_iota(jnp.int32, (UPAD_, lq), 1)
    oh_idx = (lanes_u == idxcol).astype(jnp.float32)
    rows = jax.lax.broadcasted_iota(jnp.int32, (UPAD_, lq), 0)
    lanes = jax.lax.broadcasted_iota(jnp.int32, (1, lq), 1)

    for off in (0, DK_):
        q = q_ref[0][:, off:off + DK_]
        k = k_ref[0][:, off:off + DK_]
        v = v_ref[0][:, off:off + DK_]

        ksamp = jnp.dot(oh_idx, k, preferred_element_type=jnp.float32)
        st = jax.lax.dot_general(
            ksamp, q, (((1,), (1,)), ((), ())),
            preferred_element_type=jnp.float32) * SCALE_
        smax = jnp.max(jnp.where(rows < u, st, -jnp.inf), axis=0, keepdims=True)
        smean = (jnp.sum(jnp.where(rows < u, st, 0.0), axis=0, keepdims=True)
                 * (1.0 / u))
        m = smax - smean

        m_last = jnp.max(jnp.where(lanes == lq - 1, m, -jnp.inf))
        n_gt = jnp.sum((m > m_last).astype(jnp.float32))
        n_eq_before = jnp.sum(
            jnp.logical_and(m == m_last, lanes < lq - 1).astype(jnp.float32))
        sel = (n_gt + n_eq_before) < u

        qlast = q[lq - 1:lq, :]
        s = jax.lax.dot_general(
            qlast, k, (((1,), (1,)), ((), ())),
            preferred_element_type=jnp.float32) * SCALE_
        s = s - jnp.max(s)
        e = jnp.exp(s)
        arow = jnp.dot(e / jnp.sum(e), v, preferred_element_type=jnp.float32)
        mv = jnp.sum(v, axis=0, keepdims=True) * (1.0 / lq)
        o_ref[0, :, off:off + DK_] = jnp.where(sel, arow, mv)


def _attn_last(qkv, idx_k, u):
    Bq, Lq, _ = qkv.shape
    hp = H_ // 2
    idx_pad = jnp.full((UPAD_, 1), -1, jnp.int32).at[:u, 0].set(idx_k)
    fn = functools.partial(_attn_last_kernel, u=u, lq=Lq)
    return pl.pallas_call(
        fn,
        grid=(Bq, hp),
        in_specs=[
            pl.BlockSpec((UPAD_, 1), lambda b, h: (0, 0)),
            pl.BlockSpec((1, Lq, 2 * DK_), lambda b, h: (b, 0, h)),
            pl.BlockSpec((1, Lq, 2 * DK_), lambda b, h: (b, 0, hp + h)),
            pl.BlockSpec((1, Lq, 2 * DK_), lambda b, h: (b, 0, 2 * hp + h)),
        ],
        out_specs=pl.BlockSpec((1, 1, 2 * DK_), lambda b, h: (b, 0, h)),
        out_shape=jax.ShapeDtypeStruct((Bq, 1, D_), jnp.float32),
    )(idx_pad, qkv, qkv, qkv)


def _tail_kernel(c_ref, r_ref, wo_ref, bo_ref, g1_ref, be1_ref,
                 w1_ref, b1_ref, w2_ref, b2_ref, g2_ref, be2_ref,
                 wf_ref, bf_ref, o_ref):
    y = (jnp.dot(c_ref[...], wo_ref[...], preferred_element_type=jnp.float32)
         + bo_ref[...] + r_ref[...])
    mu = jnp.mean(y, axis=-1, keepdims=True)
    d = y - mu
    va = jnp.mean(d * d, axis=-1, keepdims=True)
    h1 = d / jnp.sqrt(va + 1e-5) * g1_ref[...] + be1_ref[...]
    t = jnp.dot(h1, w1_ref[...], preferred_element_type=jnp.float32) + b1_ref[...]
    t = 0.5 * t * (1.0 + jax.lax.erf(t * (1.0 / math.sqrt(2.0))))
    y2 = (jnp.dot(t, w2_ref[...], preferred_element_type=jnp.float32)
          + b2_ref[...] + h1)
    mu2 = jnp.mean(y2, axis=-1, keepdims=True)
    d2 = y2 - mu2
    va2 = jnp.mean(d2 * d2, axis=-1, keepdims=True)
    h2 = d2 / jnp.sqrt(va2 + 1e-5) * g2_ref[...] + be2_ref[...]
    o_ref[...] = (jnp.dot(h2, wf_ref[...], preferred_element_type=jnp.float32)
                  + bf_ref[...])


def _tail(ctx_last, h_last, lp, wf, bf):
    Bx = ctx_last.shape[0]
    return pl.pallas_call(
        _tail_kernel,
        out_shape=jax.ShapeDtypeStruct((Bx, wf.shape[1]), jnp.float32),
    )(ctx_last, h_last, lp["Wo"], lp["bo"].reshape(1, -1),
      lp["g1"].reshape(1, -1), lp["be1"].reshape(1, -1),
      lp["W1"], lp["b1"].reshape(1, -1), lp["W2"], lp["b2"].reshape(1, -1),
      lp["g2"].reshape(1, -1), lp["be2"].reshape(1, -1),
      wf, bf.reshape(1, -1))


def _distill_kernel(eo_ref, w_ref, b_ref, out_ref):
    e = eo_ref[0][:, :D_]
    od = eo_ref[0][:, D_:]
    bc = b_ref[...]
    dd = lambda a, t: jnp.dot(a, w_ref[t], preferred_element_type=jnp.float32)
    a0 = dd(od, 0)
    ce = (jnp.concatenate([jnp.zeros((1, a0.shape[1]), a0.dtype), a0[:-1]],
                          axis=0)
          + dd(e, 1) + dd(od, 2) + bc)
    b2 = dd(e, 2)
    co = (dd(e, 0) + dd(od, 1)
          + jnp.concatenate([b2[1:], jnp.zeros((1, b2.shape[1]), b2.dtype)],
                            axis=0)
          + bc)
    ce = jnp.where(ce > 0, ce, jnp.exp(jnp.minimum(ce, 0.0)) - 1.0)
    co = jnp.where(co > 0, co, jnp.exp(jnp.minimum(co, 0.0)) - 1.0)
    out_ref[0] = jnp.maximum(ce, co)


def _distill(eo, wt, bc):
    Bx, Lh, D2 = eo.shape
    D = D2 // 2
    return pl.pallas_call(
        _distill_kernel,
        grid=(Bx,),
        in_specs=[
            pl.BlockSpec((1, Lh, D2), lambda b_: (b_, 0, 0)),
            pl.BlockSpec((3, D, D), lambda b_: (0, 0, 0)),
            pl.BlockSpec((1, D), lambda b_: (0, 0)),
        ],
        out_specs=pl.BlockSpec((1, Lh, D), lambda b_: (b_, 0, 0)),
        out_shape=jax.ShapeDtypeStruct((Bx, Lh, D), jnp.float32),
    )(eo, wt, bc)


def _final_kernel(a_ref, w_ref, b_ref, o_ref):
    o_ref[...] = (
        jnp.dot(a_ref[...], w_ref[...], preferred_element_type=jnp.float32)
        + b_ref[...]
    )


def _final(a, w, b):
    Bx = a.shape[0]
    return pl.pallas_call(
        _final_kernel,
        out_shape=jax.ShapeDtypeStruct((Bx, w.shape[1]), jnp.float32),
    )(a, w, b)


def kernel(x, params):
    p = params
    Bx, Lx, _ = x.shape
    layers = p["layers"]
    pos = p["pos"][:Lx]
    wqkvs = [jnp.concatenate([lp["Wq"], lp["Wk"], lp["Wv"]], axis=1)
             for lp in layers]
    bqkvs = [jnp.concatenate([lp["bq"], lp["bk"], lp["bv"]]).reshape(1, -1)
             for lp in layers]
    Lq = Lx
    hf = None
    qkvf = None
    for i, lp in enumerate(layers):
        if i == 0:
            hf, qkvf = _embed_qkv(x.reshape(Bx * Lx, -1), p["W_in"],
                                  p["b_in"].reshape(1, -1), pos,
                                  wqkvs[0], bqkvs[0])
        elif qkvf is None:
            qkvf = _mm(hf, wqkvs[i], bqkvs[i])
        qkv3 = qkvf.reshape(Bx, Lq, 3 * D_)
        u = max(1, min(FACTOR_ * math.ceil(math.log(Lq + 1)), Lq))
        rkey = jax.random.fold_in(jax.random.key(7), i)
        idx_k = jax.random.permutation(rkey, Lq)[:u].astype(jnp.int32)
        if i == len(layers) - 1 and i >= len(p["distill"]):
            ctx_last = _attn_last(qkv3, idx_k, u).reshape(Bx, D_)
            h_last = hf.reshape(Bx, Lq, D_)[:, -1, :]
            return _tail(ctx_last, h_last, lp, p["W_fc"], p["b_fc"])
        ctx = _attn(qkv3, idx_k, u)
        h2 = _block(ctx.reshape(Bx * Lq, D_), hf, lp)
        hf = h2
        qkvf = None
        if i < len(p["distill"]):
            dp = p["distill"][i]
            wt = jnp.transpose(dp["Wc"], (2, 1, 0))
            hd = _distill(h2.reshape(Bx, Lq // 2, 2 * D_), wt,
                          dp["bc"].reshape(1, -1))
            Lq = Lq // 2
            hf = hd.reshape(Bx * Lq, D_)
    h3 = hf.reshape(Bx, Lq, D_)
    return _final(h3[:, -1, :], p["W_fc"], p["b_fc"].reshape(1, -1))


# speedup vs baseline: 1.0021x; 1.0021x over previous
import functools
import math

import jax
import jax.numpy as jnp
from jax.experimental import pallas as pl
from jax.experimental.pallas import tpu as pltpu

D_, H_, DFF_, PRED_ = 768, 12, 2048, 24
FACTOR_ = 5
DK_ = D_ // H_
SCALE_ = 1.0 / math.sqrt(DK_)
UPAD_ = 40


def _split(w):
    w_hi = w.astype(jnp.bfloat16)
    w_lo = (w - w_hi.astype(jnp.float32)).astype(jnp.bfloat16)
    return w_hi, w_lo


def _d3(ah, al, wh, wl):
    d = lambda p, q: jnp.dot(p, q, preferred_element_type=jnp.float32)
    return d(ah, wh) + (d(ah, wl) + d(al, wh))


def _asplit(a):
    ah = a.astype(jnp.bfloat16)
    al = (a - ah.astype(jnp.float32)).astype(jnp.bfloat16)
    return ah, al


def _embed_qkv_kernel(x_ref, w_ref, b_ref, pos_ref, wq_ref, bq_ref,
                      h_ref, qkv_ref):
    h = (jnp.dot(x_ref[...], w_ref[...], preferred_element_type=jnp.float32)
         + b_ref[...] + pos_ref[...])
    h_ref[...] = h
    qkv_ref[...] = (jnp.dot(h, wq_ref[...], preferred_element_type=jnp.float32)
                    + bq_ref[...])


def _embed_qkv(x2, w, b, pos, wqkv, bqkv, tm=512):
    M, F = x2.shape
    D = w.shape[1]
    N = wqkv.shape[1]
    tm = min(tm, pos.shape[0])
    nl = pos.shape[0] // tm
    return pl.pallas_call(
        _embed_qkv_kernel,
        grid=(M // tm,),
        in_specs=[
            pl.BlockSpec((tm, F), lambda m: (m, 0)),
            pl.BlockSpec((F, D), lambda m: (0, 0)),
            pl.BlockSpec((1, D), lambda m: (0, 0)),
            pl.BlockSpec((tm, D), lambda m: (m % nl, 0)),
            pl.BlockSpec((D, N), lambda m: (0, 0)),
            pl.BlockSpec((1, N), lambda m: (0, 0)),
        ],
        out_specs=[
            pl.BlockSpec((tm, D), lambda m: (m, 0)),
            pl.BlockSpec((tm, N), lambda m: (m, 0)),
        ],
        out_shape=[
            jax.ShapeDtypeStruct((M, D), jnp.float32),
            jax.ShapeDtypeStruct((M, N), jnp.float32),
        ],
    )(x2, w, b, pos, wqkv, bqkv)


def _mm_kernel(a_ref, w_ref, b_ref, o_ref):
    o_ref[...] = (
        jnp.dot(a_ref[...], w_ref[...], preferred_element_type=jnp.float32)
        + b_ref[...])


def _mm(a, w, b, tm=512):
    M, K = a.shape
    N = w.shape[1]
    tm = min(tm, M)
    return pl.pallas_call(
        _mm_kernel,
        grid=(M // tm,),
        in_specs=[
            pl.BlockSpec((tm, K), lambda m: (m, 0)),
            pl.BlockSpec((K, N), lambda m: (0, 0)),
            pl.BlockSpec((1, N), lambda m: (0, 0)),
        ],
        out_specs=pl.BlockSpec((tm, N), lambda m: (m, 0)),
        out_shape=jax.ShapeDtypeStruct((M, N), jnp.float32),
    )(a, w, b)


def _block_kernel(c_ref, r_ref, wo_ref, bo_ref, g1_ref, be1_ref,
                  w1_ref, b1_ref, w2_ref, b2_ref, g2_ref, be2_ref, o_ref):
    y = (jnp.dot(c_ref[...], wo_ref[...], preferred_element_type=jnp.float32)
         + bo_ref[...] + r_ref[...])
    mu = jnp.mean(y, axis=-1, keepdims=True)
    d = y - mu
    va = jnp.mean(d * d, axis=-1, keepdims=True)
    h1 = d / jnp.sqrt(va + 1e-5) * g1_ref[...] + be1_ref[...]
    t = (jnp.dot(h1, w1_ref[...], preferred_element_type=jnp.float32)
         + b1_ref[...])
    t = 0.5 * t * (1.0 + jax.lax.erf(t * (1.0 / math.sqrt(2.0))))
    y2 = (jnp.dot(t, w2_ref[...], preferred_element_type=jnp.float32)
          + b2_ref[...] + h1)
    mu2 = jnp.mean(y2, axis=-1, keepdims=True)
    d2 = y2 - mu2
    va2 = jnp.mean(d2 * d2, axis=-1, keepdims=True)
    o_ref[...] = d2 / jnp.sqrt(va2 + 1e-5) * g2_ref[...] + be2_ref[...]


def _block(c, r, lp, tm=512):
    M, K = c.shape
    N = lp["W1"].shape[1]
    tm = min(tm, M)
    full = lambda m: (0, 0)
    row = lambda m: (m, 0)
    return pl.pallas_call(
        _block_kernel,
        grid=(M // tm,),
        in_specs=[
            pl.BlockSpec((tm, K), row),
            pl.BlockSpec((tm, K), row),
            pl.BlockSpec((K, K), full),
            pl.BlockSpec((1, K), full),
            pl.BlockSpec((1, K), full),
            pl.BlockSpec((1, K), full),
            pl.BlockSpec((K, N), full),
            pl.BlockSpec((1, N), full),
            pl.BlockSpec((N, K), full),
            pl.BlockSpec((1, K), full),
            pl.BlockSpec((1, K), full),
            pl.BlockSpec((1, K), full),
        ],
        out_specs=pl.BlockSpec((tm, K), row),
        out_shape=jax.ShapeDtypeStruct((M, K), jnp.float32),
    )(c, r, lp["Wo"], lp["bo"].reshape(1, -1),
      lp["g1"].reshape(1, -1), lp["be1"].reshape(1, -1),
      lp["W1"], lp["b1"].reshape(1, -1), lp["W2"], lp["b2"].reshape(1, -1),
      lp["g2"].reshape(1, -1), lp["be2"].reshape(1, -1))


OHP_ = 48


def _select_kernel(idx_ref, q_ref, k_ref, oh_ref, m_ref, *, u, lq):
    idxcol = idx_ref[...]
    lanes_u = jax.lax.broadcasted_iota(jnp.int32, (UPAD_, lq), 1)
    oh_idx = jnp.where(lanes_u == idxcol, SCALE_, 0.0)

    for h in range(H_):
        q = q_ref[0][:, h * DK_:(h + 1) * DK_]
        k = k_ref[0][:, h * DK_:(h + 1) * DK_]
        ksamp = jnp.dot(oh_idx, k, preferred_element_type=jnp.float32)
        st = jax.lax.dot_general(
            ksamp, q, (((1,), (1,)), ((), ())),
            preferred_element_type=jnp.float32)
        if u < UPAD_:
            rows = jax.lax.broadcasted_iota(jnp.int32, (UPAD_, lq), 0)
            smax = jnp.max(jnp.where(rows < u, st, -jnp.inf), 0, keepdims=True)
            smean = (jnp.sum(jnp.where(rows < u, st, 0.0), 0, keepdims=True)
                     * (1.0 / u))
        else:
            smax = jnp.max(st, axis=0, keepdims=True)
            smean = jnp.sum(st, axis=0, keepdims=True) * (1.0 / u)
        m_ref[h:h + 1, :] = smax - smean

    m_ref[H_:, :] = jnp.full((16 - H_, lq), -jnp.inf, jnp.float32)
    mall = m_ref[...]
    lanes16 = jax.lax.broadcasted_iota(jnp.int32, (16, lq), 1)

    for j in range(u):
        mx = jnp.max(mall, axis=1, keepdims=True)
        i = jnp.min(jnp.where(mall == mx, lanes16, lq), axis=1, keepdims=True)
        ohj = lanes16 == i
        fj = ohj.astype(jnp.float32)
        for h in range(H_):
            oh_ref[0, h, j:j + 1, :] = fj[h:h + 1, :]
        mall = jnp.where(ohj, -jnp.inf, mall)

    ones_row = jnp.ones((1, lq), jnp.float32)
    zeros_tail = jnp.zeros((OHP_ - u - 1, lq), jnp.float32)
    for h in range(H_):
        oh_ref[0, h, u:u + 1, :] = ones_row
        oh_ref[0, h, u + 1:, :] = zeros_tail


def _apply_kernel(q_ref, k_ref, v_ref, oh_ref, o_ref, *, u, lq):
    for t, off in enumerate((0, DK_)):
        q = q_ref[0][:, off:off + DK_]
        k = k_ref[0][:, off:off + DK_]
        v = v_ref[0][:, off:off + DK_]
        oh = oh_ref[0, t]

        qtop = jnp.dot(oh, q, preferred_element_type=jnp.float32) * SCALE_
        s = jax.lax.dot_general(
            qtop, k, (((1,), (1,)), ((), ())),
            preferred_element_type=jnp.float32)
        s = s - jnp.max(s, axis=1, keepdims=True)
        e = jnp.exp(s)
        denom = jnp.dot(e, jnp.ones((lq, 1), jnp.float32),
                        preferred_element_type=jnp.float32)
        p = e / denom
        ctx_top = jnp.dot(p, v, preferred_element_type=jnp.float32)

        mv = jnp.dot(jnp.full((1, lq), 1.0 / lq, jnp.float32), v,
                     preferred_element_type=jnp.float32)
        rows_c = jax.lax.broadcasted_iota(jnp.int32, (OHP_, DK_), 0)
        ctx_aug = jnp.where(rows_c == u, mv, ctx_top - mv)
        o_ref[0, :, off:off + DK_] = jax.lax.dot_general(
            oh, ctx_aug, (((0,), (0,)), ((), ())),
            preferred_element_type=jnp.float32)


def _attn(qkv, idx_k, u):
    Bq, Lq, _ = qkv.shape
    hp = H_ // 2
    idx_pad = jnp.full((UPAD_, 1), -1, jnp.int32).at[:u, 0].set(idx_k)
    oh = pl.pallas_call(
        functools.partial(_select_kernel, u=u, lq=Lq),
        grid=(Bq,),
        in_specs=[
            pl.BlockSpec((UPAD_, 1), lambda b: (0, 0)),
            pl.BlockSpec((1, Lq, D_), lambda b: (b, 0, 0)),
            pl.BlockSpec((1, Lq, D_), lambda b: (b, 0, 1)),
        ],
        out_specs=pl.BlockSpec((1, H_, OHP_, Lq), lambda b: (b, 0, 0, 0)),
        scratch_shapes=[pltpu.VMEM((16, Lq), jnp.float32)],
        out_shape=jax.ShapeDtypeStruct((Bq, H_, OHP_, Lq), jnp.float32),
    )(idx_pad, qkv, qkv)
    return pl.pallas_call(
        functools.partial(_apply_kernel, u=u, lq=Lq),
        grid=(Bq, hp),
        in_specs=[
            pl.BlockSpec((1, Lq, 2 * DK_), lambda b, h: (b, 0, h)),
            pl.BlockSpec((1, Lq, 2 * DK_), lambda b, h: (b, 0, hp + h)),
            pl.BlockSpec((1, Lq, 2 * DK_), lambda b, h: (b, 0, 2 * hp + h)),
            pl.BlockSpec((1, 2, OHP_, Lq), lambda b, h: (b, h, 0, 0)),
        ],
        out_specs=pl.BlockSpec((1, Lq, 2 * DK_), lambda b, h: (b, 0, h)),
        out_shape=jax.ShapeDtypeStruct((Bq, Lq, D_), jnp.float32),
    )(qkv, qkv, qkv, oh)


def _attn_last_kernel(idx_ref, q_ref, k_ref, v_ref, o_ref, *, u, lq):
    idxcol = idx_ref[...]
    lanes_u = jax.lax.broadcasted_iota(jnp.int32, (UPAD_, lq), 1)
    oh_idx = (lanes_u == idxcol).astype(jnp.float32)
    rows = jax.lax.broadcasted_iota(jnp.int32, (UPAD_, lq), 0)
    lanes = jax.lax.broadcasted_iota(jnp.int32, (1, lq), 1)

    for off in (0, DK_):
        q = q_ref[0][:, off:off + DK_]
        k = k_ref[0][:, off:off + DK_]
        v = v_ref[0][:, off:off + DK_]

        ksamp = jnp.dot(oh_idx, k, preferred_element_type=jnp.float32)
        st = jax.lax.dot_general(
            ksamp, q, (((1,), (1,)), ((), ())),
            preferred_element_type=jnp.float32) * SCALE_
        smax = jnp.max(jnp.where(rows < u, st, -jnp.inf), axis=0, keepdims=True)
        smean = (jnp.sum(jnp.where(rows < u, st, 0.0), axis=0, keepdims=True)
                 * (1.0 / u))
        m = smax - smean

        m_last = jnp.max(jnp.where(lanes == lq - 1, m, -jnp.inf))
        n_gt = jnp.sum((m > m_last).astype(jnp.float32))
        n_eq_before = jnp.sum(
            jnp.logical_and(m == m_last, lanes < lq - 1).astype(jnp.float32))
        sel = (n_gt + n_eq_before) < u

        qlast = q[lq - 1:lq, :]
        s = jax.lax.dot_general(
            qlast, k, (((1,), (1,)), ((), ())),
            preferred_element_type=jnp.float32) * SCALE_
        s = s - jnp.max(s)
        e = jnp.exp(s)
        arow = jnp.dot(e / jnp.sum(e), v, preferred_element_type=jnp.float32)
        mv = jnp.sum(v, axis=0, keepdims=True) * (1.0 / lq)
        o_ref[0, :, off:off + DK_] = jnp.where(sel, arow, mv)


def _attn_last(qkv, idx_k, u):
    Bq, Lq, _ = qkv.shape
    hp = H_ // 2
    idx_pad = jnp.full((UPAD_, 1), -1, jnp.int32).at[:u, 0].set(idx_k)
    fn = functools.partial(_attn_last_kernel, u=u, lq=Lq)
    return pl.pallas_call(
        fn,
        grid=(Bq, hp),
        in_specs=[
            pl.BlockSpec((UPAD_, 1), lambda b, h: (0, 0)),
            pl.BlockSpec((1, Lq, 2 * DK_), lambda b, h: (b, 0, h)),
            pl.BlockSpec((1, Lq, 2 * DK_), lambda b, h: (b, 0, hp + h)),
            pl.BlockSpec((1, Lq, 2 * DK_), lambda b, h: (b, 0, 2 * hp + h)),
        ],
        out_specs=pl.BlockSpec((1, 1, 2 * DK_), lambda b, h: (b, 0, h)),
        out_shape=jax.ShapeDtypeStruct((Bq, 1, D_), jnp.float32),
    )(idx_pad, qkv, qkv, qkv)


def _tail_kernel(c_ref, r_ref, wo_ref, bo_ref, g1_ref, be1_ref,
                 w1_ref, b1_ref, w2_ref, b2_ref, g2_ref, be2_ref,
                 wf_ref, bf_ref, o_ref):
    y = (jnp.dot(c_ref[...], wo_ref[...], preferred_element_type=jnp.float32)
         + bo_ref[...] + r_ref[...])
    mu = jnp.mean(y, axis=-1, keepdims=True)
    d = y - mu
    va = jnp.mean(d * d, axis=-1, keepdims=True)
    h1 = d / jnp.sqrt(va + 1e-5) * g1_ref[...] + be1_ref[...]
    t = jnp.dot(h1, w1_ref[...], preferred_element_type=jnp.float32) + b1_ref[...]
    t = 0.5 * t * (1.0 + jax.lax.erf(t * (1.0 / math.sqrt(2.0))))
    y2 = (jnp.dot(t, w2_ref[...], preferred_element_type=jnp.float32)
          + b2_ref[...] + h1)
    mu2 = jnp.mean(y2, axis=-1, keepdims=True)
    d2 = y2 - mu2
    va2 = jnp.mean(d2 * d2, axis=-1, keepdims=True)
    h2 = d2 / jnp.sqrt(va2 + 1e-5) * g2_ref[...] + be2_ref[...]
    o_ref[...] = (jnp.dot(h2, wf_ref[...], preferred_element_type=jnp.float32)
                  + bf_ref[...])


def _tail(ctx_last, h_last, lp, wf, bf):
    Bx = ctx_last.shape[0]
    return pl.pallas_call(
        _tail_kernel,
        out_shape=jax.ShapeDtypeStruct((Bx, wf.shape[1]), jnp.float32),
    )(ctx_last, h_last, lp["Wo"], lp["bo"].reshape(1, -1),
      lp["g1"].reshape(1, -1), lp["be1"].reshape(1, -1),
      lp["W1"], lp["b1"].reshape(1, -1), lp["W2"], lp["b2"].reshape(1, -1),
      lp["g2"].reshape(1, -1), lp["be2"].reshape(1, -1),
      wf, bf.reshape(1, -1))


def _distill_kernel(eo_ref, w_ref, b_ref, out_ref):
    e = eo_ref[0][:, :D_]
    od = eo_ref[0][:, D_:]
    bc = b_ref[...]
    dd = lambda a, t: jnp.dot(a, w_ref[t], preferred_element_type=jnp.float32)
    a0 = dd(od, 0)
    ce = (jnp.concatenate([jnp.zeros((1, a0.shape[1]), a0.dtype), a0[:-1]],
                          axis=0)
          + dd(e, 1) + dd(od, 2) + bc)
    b2 = dd(e, 2)
    co = (dd(e, 0) + dd(od, 1)
          + jnp.concatenate([b2[1:], jnp.zeros((1, b2.shape[1]), b2.dtype)],
                            axis=0)
          + bc)
    ce = jnp.where(ce > 0, ce, jnp.exp(jnp.minimum(ce, 0.0)) - 1.0)
    co = jnp.where(co > 0, co, jnp.exp(jnp.minimum(co, 0.0)) - 1.0)
    out_ref[0] = jnp.maximum(ce, co)


def _distill(eo, wt, bc):
    Bx, Lh, D2 = eo.shape
    D = D2 // 2
    return pl.pallas_call(
        _distill_kernel,
        grid=(Bx,),
        in_specs=[
            pl.BlockSpec((1, Lh, D2), lambda b_: (b_, 0, 0)),
            pl.BlockSpec((3, D, D), lambda b_: (0, 0, 0)),
            pl.BlockSpec((1, D), lambda b_: (0, 0)),
        ],
        out_specs=pl.BlockSpec((1, Lh, D), lambda b_: (b_, 0, 0)),
        out_shape=jax.ShapeDtypeStruct((Bx, Lh, D), jnp.float32),
    )(eo, wt, bc)


def _final_kernel(a_ref, w_ref, b_ref, o_ref):
    o_ref[...] = (
        jnp.dot(a_ref[...], w_ref[...], preferred_element_type=jnp.float32)
        + b_ref[...]
    )


def _final(a, w, b):
    Bx = a.shape[0]
    return pl.pallas_call(
        _final_kernel,
        out_shape=jax.ShapeDtypeStruct((Bx, w.shape[1]), jnp.float32),
    )(a, w, b)


def kernel(x, params):
    p = params
    Bx, Lx, _ = x.shape
    layers = p["layers"]
    pos = p["pos"][:Lx]
    wqkvs = [jnp.concatenate([lp["Wq"], lp["Wk"], lp["Wv"]], axis=1)
             for lp in layers]
    bqkvs = [jnp.concatenate([lp["bq"], lp["bk"], lp["bv"]]).reshape(1, -1)
             for lp in layers]
    Lq = Lx
    hf = None
    qkvf = None
    for i, lp in enumerate(layers):
        if i == 0:
            hf, qkvf = _embed_qkv(x.reshape(Bx * Lx, -1), p["W_in"],
                                  p["b_in"].reshape(1, -1), pos,
                                  wqkvs[0], bqkvs[0])
        elif qkvf is None:
            qkvf = _mm(hf, wqkvs[i], bqkvs[i])
        qkv3 = qkvf.reshape(Bx, Lq, 3 * D_)
        u = max(1, min(FACTOR_ * math.ceil(math.log(Lq + 1)), Lq))
        rkey = jax.random.fold_in(jax.random.key(7), i)
        idx_k = jax.random.permutation(rkey, Lq)[:u].astype(jnp.int32)
        if i == len(layers) - 1 and i >= len(p["distill"]):
            ctx_last = _attn_last(qkv3, idx_k, u).reshape(Bx, D_)
            h_last = hf.reshape(Bx, Lq, D_)[:, -1, :]
            return _tail(ctx_last, h_last, lp, p["W_fc"], p["b_fc"])
        ctx = _attn(qkv3, idx_k, u)
        h2 = _block(ctx.reshape(Bx * Lq, D_), hf, lp)
        hf = h2
        qkvf = None
        if i < len(p["distill"]):
            dp = p["distill"][i]
            wt = jnp.transpose(dp["Wc"], (2, 1, 0))
            hd = _distill(h2.reshape(Bx, Lq // 2, 2 * D_), wt,
                          dp["bc"].reshape(1, -1))
            Lq = Lq // 2
            hf = hd.reshape(Bx * Lq, D_)
    h3 = hf.reshape(Bx, Lq, D_)
    return _final(h3[:, -1, :], p["W_fc"], p["b_fc"].reshape(1, -1))
